# Initial kernel scaffold; baseline (speedup 1.0000x reference)
import jax
import jax.numpy as jnp
from jax.experimental import pallas as pl


def kernel(x, edge_index, efeat, W_e, b_e, bias):
    raise NotImplementedError("write your pallas kernel here")


# speedup vs baseline: 48.8841x; 48.8841x over previous
import functools

import jax
import jax.numpy as jnp
from jax import lax
from jax.experimental import pallas as pl
from jax.experimental.pallas import tpu as pltpu
from jax.experimental.pallas import tpu_sc as plsc

F = 16
NC = 2
NS = 16
NW = NC * NS
CHUNK = 128

_mesh = plsc.VectorSubcoreMesh(core_axis_name="c", subcore_axis_name="s")


def _gather_call(x, src_r, cpw):
    @functools.partial(
        pl.kernel,
        mesh=_mesh,
        out_type=jax.ShapeDtypeStruct((NW, cpw, CHUNK, F), jnp.float32),
        compiler_params=pltpu.CompilerParams(use_tc_tiling_on_sc=False),
        scratch_types=[
            pltpu.VMEM((cpw, CHUNK), jnp.int32),
            pltpu.VMEM((cpw, CHUNK, F), jnp.float32),
            pltpu.SemaphoreType.DMA,
        ],
    )
    def gather_kernel(x_hbm, src_hbm, h_hbm, idx_v, rows_v, sem):
        cid = lax.axis_index("c")
        sid = lax.axis_index("s")
        wid = sid * NC + cid
        pltpu.sync_copy(src_hbm.at[wid], idx_v)

        def fire(j, carry):
            pltpu.make_async_copy(x_hbm.at[idx_v.at[j]], rows_v.at[j], sem).start()
            return carry

        lax.fori_loop(0, cpw, fire, 0)

        def drain(j, carry):
            pltpu.make_async_copy(x_hbm.at[idx_v.at[j]], rows_v.at[j], sem).wait()
            return carry

        lax.fori_loop(0, cpw, drain, 0)
        pltpu.sync_copy(rows_v, h_hbm.at[wid])

    return gather_kernel(x, src_r)


def _msg_call(ef_p, h, R, T, W2, B2, e_pad):
    TB = 2048
    grid = (e_pad // TB,)

    def body(ef_ref, h_ref, r_ref, t_ref, w2_ref, b2_ref, out_ref):
        efr = jnp.dot(ef_ref[...], r_ref[...], preferred_element_type=jnp.float32)
        ht = jnp.dot(h_ref[...], t_ref[...], preferred_element_type=jnp.float32)
        msg = jnp.dot(efr * ht, w2_ref[...], preferred_element_type=jnp.float32)
        msg = msg + jnp.dot(h_ref[...], b2_ref[...], preferred_element_type=jnp.float32)
        out_ref[...] = msg

    return pl.pallas_call(
        body,
        grid=grid,
        in_specs=[
            pl.BlockSpec((TB, F), lambda i: (i, 0)),
            pl.BlockSpec((TB, F), lambda i: (i, 0)),
            pl.BlockSpec((F, F * F), lambda i: (0, 0)),
            pl.BlockSpec((F, F * F), lambda i: (0, 0)),
            pl.BlockSpec((F * F, F), lambda i: (0, 0)),
            pl.BlockSpec((F, F), lambda i: (0, 0)),
        ],
        out_specs=pl.BlockSpec((TB, F), lambda i: (i, 0)),
        out_shape=jax.ShapeDtypeStruct((e_pad, F), jnp.float32),
    )(ef_p, h, R, T, W2, B2)


def _scatter_call(msg_r, dst_r, zeros, ones, cpw, n_acc):
    rows_per_tile = n_acc // NS

    @functools.partial(
        pl.kernel,
        mesh=_mesh,
        out_type=(
            jax.ShapeDtypeStruct((NC, n_acc, F), jnp.float32),
            jax.ShapeDtypeStruct((NC, n_acc, F), jnp.float32),
        ),
        compiler_params=pltpu.CompilerParams(use_tc_tiling_on_sc=False),
        scratch_types=[
            pltpu.VMEM((cpw, CHUNK), jnp.int32),
            pltpu.VMEM((cpw, CHUNK, F), jnp.float32),
            pltpu.VMEM((CHUNK, F), jnp.float32),
            pltpu.VMEM_SHARED((n_acc, F), jnp.float32),
            pltpu.VMEM_SHARED((n_acc, F), jnp.float32),
        ],
    )
    def scatter_kernel(msg_hbm, dst_hbm, zeros_hbm, ones_hbm, sum_out, cnt_out,
                       idx_v, msg_v, ones_v, acc_s, cnt_s):
        cid = lax.axis_index("c")
        sid = lax.axis_index("s")
        wid = sid * NC + cid
        r0 = sid * rows_per_tile
        rows = pl.ds(r0, rows_per_tile)
        pltpu.sync_copy(zeros_hbm.at[rows], acc_s.at[rows])
        pltpu.sync_copy(zeros_hbm.at[rows], cnt_s.at[rows])
        pltpu.sync_copy(dst_hbm.at[wid], idx_v)
        pltpu.sync_copy(msg_hbm.at[wid], msg_v)
        pltpu.sync_copy(ones_hbm, ones_v)
        plsc.subcore_barrier()

        def body(j, carry):
            pltpu.sync_copy(msg_v.at[j], acc_s.at[idx_v.at[j]], add=True)
            pltpu.sync_copy(ones_v, cnt_s.at[idx_v.at[j]], add=True)
            return carry

        lax.fori_loop(0, cpw, body, 0)
        plsc.subcore_barrier()
        pltpu.sync_copy(acc_s.at[rows], sum_out.at[cid, rows])
        pltpu.sync_copy(cnt_s.at[rows], cnt_out.at[cid, rows])

    return scatter_kernel(msg_r, dst_r, zeros, ones)


def _finalize_call(sums, cnts, bias2d, n_acc):
    def body(s_ref, c_ref, b_ref, o_ref):
        s = s_ref[0] + s_ref[1]
        c = c_ref[0] + c_ref[1]
        o_ref[...] = s / jnp.maximum(c, 1.0) + b_ref[...]

    return pl.pallas_call(
        body,
        out_shape=jax.ShapeDtypeStruct((n_acc, F), jnp.float32),
    )(sums, cnts, bias2d)


def kernel(x, edge_index, efeat, W_e, b_e, bias):
    n = x.shape[0]
    e = edge_index.shape[1]
    cpw = -(-e // (NW * CHUNK))
    e_pad = NW * cpw * CHUNK
    pad = e_pad - e
    n_acc = ((n // CHUNK) + 1) * CHUNK

    src = edge_index[0]
    dst = edge_index[1]
    src_r = jnp.concatenate([src, jnp.zeros((pad,), jnp.int32)]).reshape(NW, cpw, CHUNK)
    dst_r = jnp.concatenate([dst, jnp.full((pad,), n_acc - 1, jnp.int32)]).reshape(NW, cpw, CHUNK)
    ef_p = jnp.concatenate([efeat, jnp.zeros((pad, F), jnp.float32)])

    h = _gather_call(x, src_r, cpw).reshape(e_pad, F)

    j = lax.broadcasted_iota(jnp.int32, (F, F * F), 1)
    d = lax.broadcasted_iota(jnp.int32, (F, F * F), 0)
    R = (j // F == d).astype(jnp.float32)
    T = (j % F == d).astype(jnp.float32)
    W2 = W_e.reshape(F * F, F)
    B2 = b_e.reshape(F, F)

    msg = _msg_call(ef_p, h, R, T, W2, B2, e_pad)

    zeros = jnp.zeros((n_acc, F), jnp.float32)
    ones = jnp.ones((CHUNK, F), jnp.float32)
    sums, cnts = _scatter_call(msg.reshape(NW, cpw, CHUNK, F), dst_r, zeros, ones, cpw, n_acc)

    out = _finalize_call(sums, cnts, bias.reshape(1, F), n_acc)
    return out[:n]
